# baseline (speedup 1.0000x reference)
import jax
import jax.numpy as jnp
from jax.experimental import pallas as pl
from jax.experimental.pallas import tpu as pltpu

_G = 20
_BLOCK = 20000


def _linear_kernel(x_ref, w_ref, b_ref, o_ref):
    acc = jax.lax.dot_general(
        x_ref[...], w_ref[...],
        dimension_numbers=(((1,), (1,)), ((), ())),
        preferred_element_type=jnp.float32,
    )
    acc = acc + b_ref[...]
    o_ref[...] = jnp.broadcast_to(acc[:_BLOCK // _G, :1], (_BLOCK // _G, _G * 40))


def kernel(x, W, b):
    n, k = x.shape
    c = W.shape[0]
    b2 = b.reshape(1, c)
    packed = pl.pallas_call(
        _linear_kernel,
        grid=(n // _BLOCK,),
        in_specs=[
            pl.BlockSpec((_BLOCK, k), lambda i: (i, 0)),
            pl.BlockSpec((c, k), lambda i: (0, 0)),
            pl.BlockSpec((1, c), lambda i: (0, 0)),
        ],
        out_specs=pl.BlockSpec((_BLOCK // _G, _G * c), lambda i: (i, 0)),
        out_shape=jax.ShapeDtypeStruct((n // _G, _G * c), x.dtype),
        compiler_params=pltpu.CompilerParams(
            dimension_semantics=("arbitrary",),
        ),
    )(x, W, b2)
    return packed.reshape(n, c)


# speedup vs baseline: 1.4614x; 1.4614x over previous
import jax
import jax.numpy as jnp
from jax.experimental import pallas as pl
from jax.experimental.pallas import tpu as pltpu

_BLOCK = 4000
_NBUF = 8


def _linear_kernel(x_ref, w_ref, b_ref, o_hbm, scratch, sems):
    i = pl.program_id(0)
    nsteps = pl.num_programs(0)
    slot = jax.lax.rem(i, _NBUF)

    def _copy(step, s):
        return pltpu.make_async_copy(
            scratch.at[s],
            o_hbm.at[pl.ds(step * _BLOCK, _BLOCK), :],
            sems.at[s],
        )

    @pl.when(i >= _NBUF)
    def _():
        _copy(i - _NBUF, slot).wait()

    acc = jax.lax.dot_general(
        x_ref[...], w_ref[...],
        dimension_numbers=(((1,), (1,)), ((), ())),
        preferred_element_type=jnp.float32,
    )
    scratch[slot] = acc + b_ref[...]
    _copy(i, slot).start()

    @pl.when(i == nsteps - 1)
    def _():
        for d in range(_NBUF):
            step = i - d
            @pl.when(step >= 0)
            def _():
                _copy(step, jax.lax.rem(step, _NBUF)).wait()


def kernel(x, W, b):
    n, k = x.shape
    c = W.shape[0]
    b2 = b.reshape(1, c)
    return pl.pallas_call(
        _linear_kernel,
        grid=(n // _BLOCK,),
        in_specs=[
            pl.BlockSpec((_BLOCK, k), lambda i: (i, 0)),
            pl.BlockSpec((c, k), lambda i: (0, 0)),
            pl.BlockSpec((1, c), lambda i: (0, 0)),
        ],
        out_specs=pl.BlockSpec(memory_space=pl.ANY),
        out_shape=jax.ShapeDtypeStruct((n, c), x.dtype),
        scratch_shapes=[
            pltpu.VMEM((_NBUF, _BLOCK, c), jnp.float32),
            pltpu.SemaphoreType.DMA((_NBUF,)),
        ],
        compiler_params=pltpu.CompilerParams(
            dimension_semantics=("arbitrary",),
        ),
    )(x, W, b2)


# speedup vs baseline: 1.5895x; 1.0877x over previous
import jax
import jax.numpy as jnp
from jax.experimental import pallas as pl
from jax.experimental.pallas import tpu as pltpu

_BLOCK = 10000


def _linear_kernel(x_ref, w_ref, b_ref, o_ref):
    acc = jax.lax.dot_general(
        x_ref[...], w_ref[...],
        dimension_numbers=(((1,), (1,)), ((), ())),
        preferred_element_type=jnp.float32,
    )
    o_ref[:, :40] = acc + b_ref[...]


def kernel(x, W, b):
    n, k = x.shape
    c = W.shape[0]
    b2 = b.reshape(1, c)
    return pl.pallas_call(
        _linear_kernel,
        grid=(n // _BLOCK,),
        in_specs=[
            pl.BlockSpec((_BLOCK, k), lambda i: (i, 0)),
            pl.BlockSpec((c, k), lambda i: (0, 0)),
            pl.BlockSpec((1, c), lambda i: (0, 0)),
        ],
        out_specs=pl.BlockSpec((_BLOCK, 128), lambda i: (i, 0)),
        out_shape=jax.ShapeDtypeStruct((n, c), x.dtype),
        compiler_params=pltpu.CompilerParams(
            dimension_semantics=("arbitrary",),
        ),
    )(x, W, b2)


# speedup vs baseline: 1.6135x; 1.0151x over previous
import jax
import jax.numpy as jnp
from jax.experimental import pallas as pl
from jax.experimental.pallas import tpu as pltpu

_BLOCK = 10000
_IN_BUFS = 3


def _outer_kernel(x_hbm, w_ref, b_ref, o_hbm):
    w = w_ref[...]
    bv = b_ref[...]
    n, k = x_hbm.shape
    c = w.shape[0]

    def body(x_ref, o_ref):
        acc = jax.lax.dot_general(
            x_ref[...], w,
            dimension_numbers=(((1,), (1,)), ((), ())),
            preferred_element_type=jnp.float32,
        )
        o_ref[...] = acc + bv

    pipeline = pltpu.emit_pipeline(
        body,
        grid=(n // _BLOCK,),
        in_specs=[
            pl.BlockSpec((_BLOCK, k), lambda i: (i, 0),
                         pipeline_mode=pl.Buffered(buffer_count=_IN_BUFS)),
        ],
        out_specs=[
            pl.BlockSpec((_BLOCK, c), lambda i: (i, 0)),
        ],
    )
    pipeline(x_hbm, o_hbm)


def kernel(x, W, b):
    n, k = x.shape
    c = W.shape[0]
    b2 = b.reshape(1, c)
    return pl.pallas_call(
        _outer_kernel,
        in_specs=[
            pl.BlockSpec(memory_space=pl.ANY),
            pl.BlockSpec((c, k), lambda: (0, 0)),
            pl.BlockSpec((1, c), lambda: (0, 0)),
        ],
        out_specs=pl.BlockSpec(memory_space=pl.ANY),
        out_shape=jax.ShapeDtypeStruct((n, c), x.dtype),
    )(x, W, b2)
